# baseline (speedup 1.0000x reference)
import jax
import jax.numpy as jnp
from jax import lax
from jax.experimental import pallas as pl
from jax.experimental.pallas import tpu as pltpu
from jax.experimental.pallas import tpu_sc as plsc

N = 10000
E = 640000
K = 8
S = 1500
SPAD = 1536
NW = 32
EPW = E // NW
ECH = 4000
NCH = EPW // ECH
U = 2
NIT = ECH // (16 * U)
RB = 128
NRB = SPAD // RB
NR8 = 1250


def _sort_key(g):
    bu = lax.bitcast_convert_type(g, jnp.uint32)
    return jnp.where(bu >> 31 == 1, ~bu, bu | jnp.uint32(0x80000000))


def _prep_body(w1_ref, g1_ref, z_ref, graw_ref, mask_ref, vt_ref):
    sumw = jnp.sum(w1_ref[...])
    key1 = _sort_key(g1_ref[...] - jnp.log(w1_ref[...] / sumw))

    def bs_body(_, carry):
        lo, hi = carry
        mid = lo + (hi - lo) // 2
        cnt = jnp.sum((key1 <= mid).astype(jnp.int32))
        take = cnt >= S
        return jnp.where(take, lo, mid + 1), jnp.where(take, mid, hi)

    _, thr = lax.fori_loop(
        0, 32, bs_body, (jnp.uint32(0), jnp.uint32(0xFFFFFFFF)))
    mask = (key1 <= thr).astype(jnp.float32)

    z = z_ref[...]
    ze = jnp.exp(z - jnp.max(z, axis=0, keepdims=True))
    zs = ze / jnp.sum(ze, axis=0, keepdims=True)
    ri = lax.broadcasted_iota(jnp.int32, (K, K), 0)
    ci = lax.broadcasted_iota(jnp.int32, (K, K), 1)
    eye8 = (ri == ci).astype(jnp.float32)
    gt = lax.dot_general(eye8, graw_ref[...], (((1,), (1,)), ((), ())),
                         preferred_element_type=jnp.float32)
    gs = 1.0 / (1.0 + jnp.exp(-gt))
    zgt = zs * gs
    ct = zgt / jnp.sum(zgt, axis=1, keepdims=True)
    zsm = zs * mask
    m2 = lax.dot_general(zsm, ct, (((1,), (1,)), ((), ())),
                         preferred_element_type=jnp.float32)
    vt = lax.dot_general(m2, zs, (((1,), (0,)), ((), ())),
                         preferred_element_type=jnp.float32)
    mask_ref[...] = mask
    vt_ref[...] = vt


_prep_call = pl.pallas_call(
    _prep_body,
    out_shape=[
        jax.ShapeDtypeStruct((1, N), jnp.float32),
        jax.ShapeDtypeStruct((K, N), jnp.float32),
    ],
)

_SC_MESH = plsc.VectorSubcoreMesh(core_axis_name="c", subcore_axis_name="s")
_SC_PARAMS = pltpu.CompilerParams(needs_layout_passes=False)


def _comp_body(vt_hbm, beta_hbm, mask_hbm, vsamp_hbm, bsamp_hbm,
               vt_v, beta_v, mask_v, sid_v, vs_v, bs_v):
    wid = lax.axis_index("s") * 2 + lax.axis_index("c")

    @pl.when(wid == 0)
    def _():
        pltpu.sync_copy(vt_hbm, vt_v)
        pltpu.sync_copy(beta_hbm, beta_v)
        pltpu.sync_copy(mask_hbm, mask_v)

        def comp_step(i, c):
            sels, cnts = [], []
            for u in range(4):
                mv = mask_v[pl.ds((i * 4 + u) * 16, 16)]
                sel = mv > 0.5
                sels.append(sel)
                cnts.append(plsc.all_reduce_population_count(sel)[0])
            base = c
            for u in range(4):
                ids = lax.iota(jnp.int32, 16) + (i * 4 + u) * 16
                okm = jnp.logical_and(sels[u], base < SPAD - 15)
                plsc.store_compressed(sid_v.at[pl.ds(base, 16)], ids, mask=okm)
                base = base + cnts[u]
            return base

        cnt = lax.fori_loop(0, N // 64, comp_step, jnp.int32(0))
        cnt = comp_step(jnp.int32(N // 64), cnt) if (N // 16) % 4 else cnt

        def gath_body(j, c):
            for u in range(2):
                off = (j * 2 + u) * 16
                posv = lax.iota(jnp.int32, 16) + off
                valid = posv < cnt
                idxv = jnp.where(valid, sid_v[pl.ds(off, 16)], 0)
                bk = plsc.load_gather(beta_v, [idxv])
                bs_v[pl.ds(off, 16)] = jnp.where(valid, bk, -1e9)
                for k in range(K):
                    kk = jnp.full((16,), k, jnp.int32)
                    vs_v[pl.ds(k * SPAD + off, 16)] = plsc.load_gather(
                        vt_v, [kk, idxv])
            return c

        lax.fori_loop(0, SPAD // 32, gath_body, 0)
        pltpu.sync_copy(vs_v, vsamp_hbm)
        pltpu.sync_copy(bs_v, bsamp_hbm)


_comp_call = pl.kernel(
    _comp_body,
    out_type=[
        jax.ShapeDtypeStruct((K * SPAD,), jnp.float32),
        jax.ShapeDtypeStruct((SPAD,), jnp.float32),
    ],
    mesh=_SC_MESH,
    scratch_types=[
        pltpu.VMEM((K, N), jnp.float32),
        pltpu.VMEM((N,), jnp.float32),
        pltpu.VMEM((N,), jnp.float32),
        pltpu.VMEM((SPAD,), jnp.int32),
        pltpu.VMEM((K * SPAD,), jnp.float32),
        pltpu.VMEM((SPAD,), jnp.float32),
    ],
    compiler_params=_SC_PARAMS,
)


def _sc_sqrt(x):
    b = plsc.bitcast(x, jnp.int32)
    y = plsc.bitcast((b >> 1) + jnp.int32(0x1FBD1DF5), jnp.float32)
    return 0.5 * (y + x / y)


def _edge_body(vt_hbm, beta_hbm, mask_hbm, ii_hbm, jj_hbm, acc_hbm,
               vt_v, beta_v, mask_v, ia_v, ja_v, ib_v, jb_v, si_v, sj_v, st_v,
               sia, sja, sib, sjb):
    wid = lax.axis_index("s") * 2 + lax.axis_index("c")
    tbase = wid * EPW

    def _start(buf_i, buf_j, sem_i, sem_j, ci):
        base = tbase + ci * ECH
        pltpu.make_async_copy(ii_hbm.at[pl.ds(base, ECH)], buf_i, sem_i).start()
        pltpu.make_async_copy(jj_hbm.at[pl.ds(base, ECH)], buf_j, sem_j).start()

    def _wait(buf_i, buf_j, sem_i, sem_j):
        pltpu.make_async_copy(ii_hbm.at[pl.ds(0, ECH)], buf_i, sem_i).wait()
        pltpu.make_async_copy(jj_hbm.at[pl.ds(0, ECH)], buf_j, sem_j).wait()

    U1 = 5

    def _pass1(ich, jch, step, ec):
        sels, iis, jjs, cnts = [], [], [], []
        for u in range(U1):
            off = step * (16 * U1) + u * 16
            ii = ich[pl.ds(off, 16)]
            jj = jch[pl.ds(off, 16)]
            keep = (plsc.load_gather(mask_v, [ii])
                    * plsc.load_gather(mask_v, [jj]))
            sel = keep > 0.5
            iis.append(ii)
            jjs.append(jj)
            sels.append(sel)
            cnts.append(plsc.all_reduce_population_count(sel)[0])
        base = ec
        for u in range(U1):
            plsc.store_compressed(si_v.at[pl.ds(base, 16)], iis[u],
                                  mask=sels[u])
            plsc.store_compressed(sj_v.at[pl.ds(base, 16)], jjs[u],
                                  mask=sels[u])
            base = base + cnts[u]
        return base

    def _pass2(vi, carry):
        ab, ad = carry
        off = vi * 16
        lane = lax.iota(jnp.int32, 16) + off
        ec = carry_ec[0]
        valid = lane < ec
        ii = jnp.where(valid, si_v[pl.ds(off, 16)], 0)
        jj = jnp.where(valid, sj_v[pl.ds(off, 16)], 0)
        bsum = plsc.load_gather(beta_v, [ii]) + plsc.load_gather(beta_v, [jj])
        sq = []
        for k in range(K):
            kk = jnp.full((16,), k, jnp.int32)
            d = (plsc.load_gather(vt_v, [kk, ii])
                 - plsc.load_gather(vt_v, [kk, jj]) + 1e-6)
            sq.append(d * d)
        while len(sq) > 1:
            sq = [x + y for x, y in zip(sq[0::2], sq[1::2])]
        vf = valid.astype(jnp.float32)
        return ab + vf * bsum, ad + vf * _sc_sqrt(sq[0])

    def _chunk(ich, jch, carry):
        ec = lax.fori_loop(0, ECH // (16 * U1),
                           lambda it, c: _pass1(ich, jch, it, c),
                           jnp.int32(0))
        carry_ec[0] = ec
        nv2 = (ec + 15) // 16
        return lax.fori_loop(0, nv2, _pass2, carry)

    carry_ec = [jnp.int32(0)]
    _start(ia_v, ja_v, sia, sja, 0)
    pltpu.sync_copy(vt_hbm, vt_v)
    pltpu.sync_copy(beta_hbm, beta_v)
    pltpu.sync_copy(mask_hbm, mask_v)
    a0 = jnp.zeros((16,), jnp.float32)
    carry = (a0, a0)
    for ci in range(NCH):
        cur_i, cur_j = (ia_v, ja_v) if ci % 2 == 0 else (ib_v, jb_v)
        csi, csj = (sia, sja) if ci % 2 == 0 else (sib, sjb)
        nxt_i, nxt_j = (ib_v, jb_v) if ci % 2 == 0 else (ia_v, ja_v)
        nsi, nsj = (sib, sjb) if ci % 2 == 0 else (sia, sja)
        _wait(cur_i, cur_j, csi, csj)
        if ci + 1 < NCH:
            _start(nxt_i, nxt_j, nsi, nsj, ci + 1)
        carry = _chunk(cur_i, cur_j, carry)
    accb, accd = carry
    st_v[pl.ds(0, 16)] = accb
    st_v[pl.ds(16, 16)] = accd
    pltpu.sync_copy(st_v, acc_hbm.at[pl.ds(wid * 32, 32)])


_edge_call = pl.kernel(
    _edge_body,
    out_type=jax.ShapeDtypeStruct((NW * 32,), jnp.float32),
    mesh=_SC_MESH,
    scratch_types=[
        pltpu.VMEM((K, N), jnp.float32),
        pltpu.VMEM((N,), jnp.float32),
        pltpu.VMEM((N,), jnp.float32),
        pltpu.VMEM((ECH,), jnp.int32),
        pltpu.VMEM((ECH,), jnp.int32),
        pltpu.VMEM((ECH,), jnp.int32),
        pltpu.VMEM((ECH,), jnp.int32),
        pltpu.VMEM((ECH,), jnp.int32),
        pltpu.VMEM((ECH,), jnp.int32),
        pltpu.VMEM((32,), jnp.float32),
        pltpu.SemaphoreType.DMA,
        pltpu.SemaphoreType.DMA,
        pltpu.SemaphoreType.DMA,
        pltpu.SemaphoreType.DMA,
    ],
    compiler_params=_SC_PARAMS,
)


def _dense_body(vs_ref, bs_ref, a_ref, z1_ref):
    i = pl.program_id(0)
    at_full = vs_ref[...]
    bs = bs_ref[...]
    atr = vs_ref[:, pl.ds(i * RB, RB)]
    bsr = bs_ref[:, pl.ds(i * RB, RB)]

    ones_t = jnp.ones((1, SPAD), jnp.float32)
    cdims = (((0,), (0,)), ((), ()))
    nt = jnp.sum(at_full * at_full, axis=0, keepdims=True)
    rt = jnp.sum(at_full, axis=0, keepdims=True)
    nr = lax.dot_general(jnp.sum(atr * atr, axis=0, keepdims=True), ones_t,
                         cdims, preferred_element_type=jnp.float32)
    rr = lax.dot_general(jnp.sum(atr, axis=0, keepdims=True), ones_t,
                         cdims, preferred_element_type=jnp.float32)
    br = lax.dot_general(bsr, ones_t, cdims,
                         preferred_element_type=jnp.float32)
    p = lax.dot_general(atr, at_full, cdims,
                        preferred_element_type=jnp.float32)

    a = a_ref[0]
    sa = jnp.maximum(a, 0.0) + jnp.log(1.0 + jnp.exp(-jnp.abs(a)))
    d2 = nr + nt - 2.0 * p + 2e-6 * (rr - rt) + 8e-12
    dist = jnp.sqrt(jnp.maximum(d2, 0.0))
    mat = jnp.exp(br + bs - sa * dist)
    rowi = lax.broadcasted_iota(jnp.int32, (RB, SPAD), 0) + i * RB
    coli = lax.broadcasted_iota(jnp.int32, (RB, SPAD), 1)
    s_off = jnp.sum(jnp.where(rowi == coli, 0.0, mat))
    e1 = jnp.exp(jnp.float32(1.0))
    part = 0.5 * (e1 * e1) * s_off

    @pl.when(i == 0)
    def _():
        z1_ref[0, 0] = part

    @pl.when(i > 0)
    def _():
        z1_ref[0, 0] = z1_ref[0, 0] + part


_dense_call = pl.pallas_call(
    _dense_body,
    grid=(NRB,),
    in_specs=[
        pl.BlockSpec((K, SPAD), lambda i: (0, 0)),
        pl.BlockSpec((1, SPAD), lambda i: (0, 0)),
        pl.BlockSpec(memory_space=pltpu.SMEM),
    ],
    out_specs=pl.BlockSpec(memory_space=pltpu.SMEM),
    out_shape=jax.ShapeDtypeStruct((1, 1), jnp.float32),
)


def _comb_body(acc_ref, z1_ref, a_ref, out_ref):
    sb = jnp.sum(acc_ref[:, 0:16])
    sd = jnp.sum(acc_ref[:, 16:32])
    a = a_ref[0]
    sa = jnp.maximum(a, 0.0) + jnp.log(1.0 + jnp.exp(-jnp.abs(a)))
    out_ref[0, 0] = (sb - sa * sd) - z1_ref[0, 0]


_comb_call = pl.pallas_call(
    _comb_body,
    in_specs=[
        pl.BlockSpec((NW, 32), lambda: (0, 0)),
        pl.BlockSpec(memory_space=pltpu.SMEM),
        pl.BlockSpec(memory_space=pltpu.SMEM),
    ],
    out_specs=pl.BlockSpec(memory_space=pltpu.SMEM),
    out_shape=jax.ShapeDtypeStruct((1, 1), jnp.float32),
)


def kernel(sampling_weights, sparse_i_idx, sparse_j_idx, beta, a, Z, G):
    negg = -jax.random.gumbel(jax.random.key(123), (1, N), jnp.float32)
    maskf, vt = _prep_call(sampling_weights.reshape(1, N), negg, Z, G)
    mask1 = maskf.reshape(N)
    vsamp, bsamp = _comp_call(vt, beta, mask1)
    acc = _edge_call(vt, beta, mask1, sparse_i_idx, sparse_j_idx)
    z1 = _dense_call(vsamp.reshape(K, SPAD), bsamp.reshape(1, SPAD), a)
    out = _comb_call(acc.reshape(NW, 32), z1, a)
    return out[0, 0]


# speedup vs baseline: 1.1106x; 1.1106x over previous
import jax
import jax.numpy as jnp
from jax import lax
from jax.experimental import pallas as pl
from jax.experimental.pallas import tpu as pltpu
from jax.experimental.pallas import tpu_sc as plsc

N = 10000
E = 640000
K = 8
S = 1500
SPAD = 1536
NW = 32
EPW = E // NW
ECH = 4000
NCH = EPW // ECH
U = 2
NIT = ECH // (16 * U)
RB = 128
NRB = SPAD // RB
NR8 = 1250


def _sort_key(g):
    bu = lax.bitcast_convert_type(g, jnp.uint32)
    return jnp.where(bu >> 31 == 1, ~bu, bu | jnp.uint32(0x80000000))


def _prep_body(w1_ref, g1_ref, z_ref, gt_ref, mask_ref, vt_ref):
    sumw = jnp.sum(w1_ref[...])
    key1 = _sort_key(g1_ref[...] - jnp.log(w1_ref[...] / sumw))

    def bs_body(_, carry):
        lo, hi = carry
        mid = lo + (hi - lo) // 2
        cnt = jnp.sum((key1 <= mid).astype(jnp.int32))
        take = cnt >= S
        return jnp.where(take, lo, mid + 1), jnp.where(take, mid, hi)

    _, thr = lax.fori_loop(
        0, 32, bs_body, (jnp.uint32(0), jnp.uint32(0xFFFFFFFF)))
    mask = (key1 <= thr).astype(jnp.float32)

    z = z_ref[...]
    ze = jnp.exp(z - jnp.max(z, axis=0, keepdims=True))
    zs = ze / jnp.sum(ze, axis=0, keepdims=True)
    gt = gt_ref[...]
    gs = 1.0 / (1.0 + jnp.exp(-gt))
    zgt = zs * gs
    ct = zgt / jnp.sum(zgt, axis=1, keepdims=True)
    zsm = zs * mask
    m2 = lax.dot_general(zsm, ct, (((1,), (1,)), ((), ())),
                         preferred_element_type=jnp.float32)
    vt = lax.dot_general(m2, zs, (((1,), (0,)), ((), ())),
                         preferred_element_type=jnp.float32)
    mask_ref[...] = mask
    vt_ref[...] = vt


_prep_call = pl.pallas_call(
    _prep_body,
    out_shape=[
        jax.ShapeDtypeStruct((1, N), jnp.float32),
        jax.ShapeDtypeStruct((K, N), jnp.float32),
    ],
)

_SC_MESH = plsc.VectorSubcoreMesh(core_axis_name="c", subcore_axis_name="s")
_SC_PARAMS = pltpu.CompilerParams(needs_layout_passes=False)


def _comp_body(vt_hbm, beta_hbm, mask_hbm, vsamp_hbm, bsamp_hbm,
               vt_v, beta_v, mask_v, sid_v, vs_v, bs_v):
    wid = lax.axis_index("s") * 2 + lax.axis_index("c")

    @pl.when(wid == 0)
    def _():
        pltpu.sync_copy(vt_hbm, vt_v)
        pltpu.sync_copy(beta_hbm, beta_v)
        pltpu.sync_copy(mask_hbm, mask_v)

        def comp_step(i, cv):
            sels, cums, pops = [], [], []
            for u in range(4):
                mv = mask_v[pl.ds((i * 4 + u) * 16, 16)]
                sel = mv > 0.5
                sels.append(sel)
                cums.append(plsc.cumsum(sel.astype(jnp.int32)))
                pops.append(plsc.all_reduce_population_count(sel))
            base = cv
            for u in range(4):
                pos = base + cums[u] - 1
                ids = lax.iota(jnp.int32, 16) + (i * 4 + u) * 16
                okm = sels[u] & (pos < SPAD)
                plsc.store_scatter(sid_v, [pos], ids, mask=okm)
                base = base + pops[u]
            return base

        cntv = lax.fori_loop(0, N // 64, comp_step, jnp.zeros((16,), jnp.int32))
        cnt = cntv[0]

        def gath_body(j, c):
            for u in range(2):
                off = (j * 2 + u) * 16
                posv = lax.iota(jnp.int32, 16) + off
                valid = posv < cnt
                idxv = jnp.where(valid, sid_v[pl.ds(off, 16)], 0)
                bk = plsc.load_gather(beta_v, [idxv])
                bs_v[pl.ds(off, 16)] = jnp.where(valid, bk, -1e9)
                for k in range(K):
                    kk = jnp.full((16,), k, jnp.int32)
                    vs_v[pl.ds(k * SPAD + off, 16)] = plsc.load_gather(
                        vt_v, [kk, idxv])
            return c

        lax.fori_loop(0, SPAD // 32, gath_body, 0)
        pltpu.sync_copy(vs_v, vsamp_hbm)
        pltpu.sync_copy(bs_v, bsamp_hbm)


_comp_call = pl.kernel(
    _comp_body,
    out_type=[
        jax.ShapeDtypeStruct((K * SPAD,), jnp.float32),
        jax.ShapeDtypeStruct((SPAD,), jnp.float32),
    ],
    mesh=_SC_MESH,
    scratch_types=[
        pltpu.VMEM((K, N), jnp.float32),
        pltpu.VMEM((N,), jnp.float32),
        pltpu.VMEM((N,), jnp.float32),
        pltpu.VMEM((SPAD,), jnp.int32),
        pltpu.VMEM((K * SPAD,), jnp.float32),
        pltpu.VMEM((SPAD,), jnp.float32),
    ],
    compiler_params=_SC_PARAMS,
)


def _sc_sqrt(x):
    b = plsc.bitcast(x, jnp.int32)
    y = plsc.bitcast((b >> 1) + jnp.int32(0x1FBD1DF5), jnp.float32)
    return 0.5 * (y + x / y)


def _edge_body(vt_hbm, beta_hbm, mask_hbm, ii_hbm, jj_hbm, acc_hbm,
               vt_v, beta_v, mask_v, ia_v, ja_v, ib_v, jb_v, si_v, sj_v, st_v,
               sia, sja, sib, sjb):
    wid = lax.axis_index("s") * 2 + lax.axis_index("c")
    tbase = wid * EPW

    def _start(buf_i, buf_j, sem_i, sem_j, ci):
        base = tbase + ci * ECH
        pltpu.make_async_copy(ii_hbm.at[pl.ds(base, ECH)], buf_i, sem_i).start()
        pltpu.make_async_copy(jj_hbm.at[pl.ds(base, ECH)], buf_j, sem_j).start()

    def _wait(buf_i, buf_j, sem_i, sem_j):
        pltpu.make_async_copy(ii_hbm.at[pl.ds(0, ECH)], buf_i, sem_i).wait()
        pltpu.make_async_copy(jj_hbm.at[pl.ds(0, ECH)], buf_j, sem_j).wait()

    U1 = 5

    def _pass1(ich, jch, step, ecv):
        sels, iis, jjs, cums, pops = [], [], [], [], []
        for u in range(U1):
            off = step * (16 * U1) + u * 16
            ii = ich[pl.ds(off, 16)]
            jj = jch[pl.ds(off, 16)]
            keep = (plsc.load_gather(mask_v, [ii])
                    * plsc.load_gather(mask_v, [jj]))
            sel = keep > 0.5
            iis.append(ii)
            jjs.append(jj)
            sels.append(sel)
            cums.append(plsc.cumsum(sel.astype(jnp.int32)))
            pops.append(plsc.all_reduce_population_count(sel))
        base = ecv
        for u in range(U1):
            pos = base + cums[u] - 1
            plsc.store_scatter(si_v, [pos], iis[u], mask=sels[u])
            plsc.store_scatter(sj_v, [pos], jjs[u], mask=sels[u])
            base = base + pops[u]
        return base

    def _pass2(vi, carry):
        ab, ad = carry
        off = vi * 16
        lane = lax.iota(jnp.int32, 16) + off
        ec = carry_ec[0]
        valid = lane < ec
        ii = jnp.where(valid, si_v[pl.ds(off, 16)], 0)
        jj = jnp.where(valid, sj_v[pl.ds(off, 16)], 0)
        bsum = plsc.load_gather(beta_v, [ii]) + plsc.load_gather(beta_v, [jj])
        sq = []
        for k in range(K):
            kk = jnp.full((16,), k, jnp.int32)
            d = (plsc.load_gather(vt_v, [kk, ii])
                 - plsc.load_gather(vt_v, [kk, jj]) + 1e-6)
            sq.append(d * d)
        while len(sq) > 1:
            sq = [x + y for x, y in zip(sq[0::2], sq[1::2])]
        vf = valid.astype(jnp.float32)
        return ab + vf * bsum, ad + vf * _sc_sqrt(sq[0])

    def _chunk(ich, jch, carry):
        ecv = lax.fori_loop(0, ECH // (16 * U1),
                            lambda it, c: _pass1(ich, jch, it, c),
                            jnp.zeros((16,), jnp.int32))
        ec = ecv[0]
        carry_ec[0] = ec
        nv2 = (ec + 15) // 16
        return lax.fori_loop(0, nv2, _pass2, carry)

    carry_ec = [jnp.int32(0)]
    _start(ia_v, ja_v, sia, sja, 0)
    pltpu.sync_copy(vt_hbm, vt_v)
    pltpu.sync_copy(beta_hbm, beta_v)
    pltpu.sync_copy(mask_hbm, mask_v)
    a0 = jnp.zeros((16,), jnp.float32)
    carry = (a0, a0)
    for ci in range(NCH):
        cur_i, cur_j = (ia_v, ja_v) if ci % 2 == 0 else (ib_v, jb_v)
        csi, csj = (sia, sja) if ci % 2 == 0 else (sib, sjb)
        nxt_i, nxt_j = (ib_v, jb_v) if ci % 2 == 0 else (ia_v, ja_v)
        nsi, nsj = (sib, sjb) if ci % 2 == 0 else (sia, sja)
        _wait(cur_i, cur_j, csi, csj)
        if ci + 1 < NCH:
            _start(nxt_i, nxt_j, nsi, nsj, ci + 1)
        carry = _chunk(cur_i, cur_j, carry)
    accb, accd = carry
    st_v[pl.ds(0, 16)] = accb
    st_v[pl.ds(16, 16)] = accd
    pltpu.sync_copy(st_v, acc_hbm.at[pl.ds(wid * 32, 32)])


_edge_call = pl.kernel(
    _edge_body,
    out_type=jax.ShapeDtypeStruct((NW * 32,), jnp.float32),
    mesh=_SC_MESH,
    scratch_types=[
        pltpu.VMEM((K, N), jnp.float32),
        pltpu.VMEM((N,), jnp.float32),
        pltpu.VMEM((N,), jnp.float32),
        pltpu.VMEM((ECH,), jnp.int32),
        pltpu.VMEM((ECH,), jnp.int32),
        pltpu.VMEM((ECH,), jnp.int32),
        pltpu.VMEM((ECH,), jnp.int32),
        pltpu.VMEM((ECH,), jnp.int32),
        pltpu.VMEM((ECH,), jnp.int32),
        pltpu.VMEM((32,), jnp.float32),
        pltpu.SemaphoreType.DMA,
        pltpu.SemaphoreType.DMA,
        pltpu.SemaphoreType.DMA,
        pltpu.SemaphoreType.DMA,
    ],
    compiler_params=_SC_PARAMS,
)


def _dense_body(vs_ref, bs_ref, a_ref, z1_ref):
    i = pl.program_id(0)
    at_full = vs_ref[...]
    bs = bs_ref[...]
    atr = vs_ref[:, pl.ds(i * RB, RB)]
    bsr = bs_ref[:, pl.ds(i * RB, RB)]

    ones_t = jnp.ones((1, SPAD), jnp.float32)
    cdims = (((0,), (0,)), ((), ()))
    nt = jnp.sum(at_full * at_full, axis=0, keepdims=True)
    rt = jnp.sum(at_full, axis=0, keepdims=True)
    nr = lax.dot_general(jnp.sum(atr * atr, axis=0, keepdims=True), ones_t,
                         cdims, preferred_element_type=jnp.float32)
    rr = lax.dot_general(jnp.sum(atr, axis=0, keepdims=True), ones_t,
                         cdims, preferred_element_type=jnp.float32)
    br = lax.dot_general(bsr, ones_t, cdims,
                         preferred_element_type=jnp.float32)
    p = lax.dot_general(atr, at_full, cdims,
                        preferred_element_type=jnp.float32)

    a = a_ref[0]
    sa = jnp.maximum(a, 0.0) + jnp.log(1.0 + jnp.exp(-jnp.abs(a)))
    d2 = nr + nt - 2.0 * p + 2e-6 * (rr - rt) + 8e-12
    dist = jnp.sqrt(jnp.maximum(d2, 0.0))
    mat = jnp.exp(br + bs - sa * dist)
    rowi = lax.broadcasted_iota(jnp.int32, (RB, SPAD), 0) + i * RB
    coli = lax.broadcasted_iota(jnp.int32, (RB, SPAD), 1)
    s_off = jnp.sum(jnp.where(rowi == coli, 0.0, mat))
    e1 = jnp.exp(jnp.float32(1.0))
    part = 0.5 * (e1 * e1) * s_off

    @pl.when(i == 0)
    def _():
        z1_ref[0, 0] = part

    @pl.when(i > 0)
    def _():
        z1_ref[0, 0] = z1_ref[0, 0] + part


_dense_call = pl.pallas_call(
    _dense_body,
    grid=(NRB,),
    in_specs=[
        pl.BlockSpec((K, SPAD), lambda i: (0, 0)),
        pl.BlockSpec((1, SPAD), lambda i: (0, 0)),
        pl.BlockSpec(memory_space=pltpu.SMEM),
    ],
    out_specs=pl.BlockSpec(memory_space=pltpu.SMEM),
    out_shape=jax.ShapeDtypeStruct((1, 1), jnp.float32),
)


def _comb_body(acc_ref, z1_ref, a_ref, out_ref):
    sb = jnp.sum(acc_ref[:, 0:16])
    sd = jnp.sum(acc_ref[:, 16:32])
    a = a_ref[0]
    sa = jnp.maximum(a, 0.0) + jnp.log(1.0 + jnp.exp(-jnp.abs(a)))
    out_ref[0, 0] = (sb - sa * sd) - z1_ref[0, 0]


_comb_call = pl.pallas_call(
    _comb_body,
    in_specs=[
        pl.BlockSpec((NW, 32), lambda: (0, 0)),
        pl.BlockSpec(memory_space=pltpu.SMEM),
        pl.BlockSpec(memory_space=pltpu.SMEM),
    ],
    out_specs=pl.BlockSpec(memory_space=pltpu.SMEM),
    out_shape=jax.ShapeDtypeStruct((1, 1), jnp.float32),
)


def kernel(sampling_weights, sparse_i_idx, sparse_j_idx, beta, a, Z, G):
    negg = -jax.random.gumbel(jax.random.key(123), (1, N), jnp.float32)
    maskf, vt = _prep_call(sampling_weights.reshape(1, N), negg, Z, G.T)
    mask1 = maskf.reshape(N)
    vsamp, bsamp = _comp_call(vt, beta, mask1)
    acc = _edge_call(vt, beta, mask1, sparse_i_idx, sparse_j_idx)
    z1 = _dense_call(vsamp.reshape(K, SPAD), bsamp.reshape(1, SPAD), a)
    out = _comb_call(acc.reshape(NW, 32), z1, a)
    return out[0, 0]


# speedup vs baseline: 1.2330x; 1.1102x over previous
import jax
import jax.numpy as jnp
from jax import lax
from jax.experimental import pallas as pl
from jax.experimental.pallas import tpu as pltpu
from jax.experimental.pallas import tpu_sc as plsc

N = 10000
E = 640000
K = 8
S = 1500
SPAD = 1536
NW = 32
EPW = E // NW
ECH = 4000
NCH = EPW // ECH
U = 2
NIT = ECH // (16 * U)
RB = 128
NRB = SPAD // RB
NR8 = 1250


def _sort_key(g):
    bu = lax.bitcast_convert_type(g, jnp.uint32)
    return jnp.where(bu >> 31 == 1, ~bu, bu | jnp.uint32(0x80000000))


def _prep_body(w1_ref, g1_ref, z_ref, gt_ref, mask_ref, vt_ref):
    sumw = jnp.sum(w1_ref[...])
    key1 = _sort_key(g1_ref[...] - jnp.log(w1_ref[...] / sumw))

    kvec = (lax.broadcasted_iota(jnp.uint32, (8, 1), 0) + jnp.uint32(1))

    flip = jnp.uint32(0x80000000)

    def _s(x):
        return lax.bitcast_convert_type(x ^ flip, jnp.int32)

    def _u(x):
        return lax.bitcast_convert_type(x, jnp.uint32) ^ flip

    def bs_body(_, carry):
        lo, hi = carry
        step = (hi - lo) // 8
        step = jnp.where(step == 0, jnp.uint32(1), step)
        raw = lo + step * kvec
        mids = _u(jnp.minimum(_s(raw), _s(hi)))
        cnts = jnp.sum((key1 <= mids).astype(jnp.int32), axis=1, keepdims=True)
        ge = cnts >= S
        hi2 = _u(jnp.min(jnp.where(ge, _s(mids), _s(hi))))
        mp1 = jnp.minimum(_s(mids + 1), _s(hi))
        lo2 = _u(jnp.max(jnp.where(ge, _s(lo), mp1)))
        return lo2, hi2

    lo, hi = lax.fori_loop(
        0, 12, bs_body, (jnp.uint32(0), jnp.uint32(0xFFFFFFFF)))
    cfin = jnp.sum((key1 <= lo).astype(jnp.int32))
    thr = jnp.where(cfin >= S, lo, hi)
    mask = (key1 <= thr).astype(jnp.float32)

    z = z_ref[...]
    ze = jnp.exp(z - jnp.max(z, axis=0, keepdims=True))
    zs = ze / jnp.sum(ze, axis=0, keepdims=True)
    gt = gt_ref[...]
    gs = 1.0 / (1.0 + jnp.exp(-gt))
    zgt = zs * gs
    ct = zgt / jnp.sum(zgt, axis=1, keepdims=True)
    zsm = zs * mask
    m2 = lax.dot_general(zsm, ct, (((1,), (1,)), ((), ())),
                         preferred_element_type=jnp.float32)
    vt = lax.dot_general(m2, zs, (((1,), (0,)), ((), ())),
                         preferred_element_type=jnp.float32)
    mask_ref[...] = mask
    vt_ref[...] = vt


_prep_call = pl.pallas_call(
    _prep_body,
    out_shape=[
        jax.ShapeDtypeStruct((1, N), jnp.float32),
        jax.ShapeDtypeStruct((K, N), jnp.float32),
    ],
)

_SC_MESH = plsc.VectorSubcoreMesh(core_axis_name="c", subcore_axis_name="s")
_SC_PARAMS = pltpu.CompilerParams(needs_layout_passes=False)


def _comp_body(vt_hbm, beta_hbm, mask_hbm, vsamp_hbm, bsamp_hbm,
               vt_v, beta_v, mask_v, sid_v, vs_v, bs_v, csem, bsem):
    wid = lax.axis_index("s") * 2 + lax.axis_index("c")

    @pl.when(wid == 0)
    def _():
        pltpu.make_async_copy(vt_hbm, vt_v, csem).start()
        pltpu.make_async_copy(beta_hbm, beta_v, bsem).start()
        pltpu.sync_copy(mask_hbm, mask_v)

        def comp_step(i, cv):
            sels, cums, pops = [], [], []
            for u in range(4):
                mv = mask_v[pl.ds((i * 4 + u) * 16, 16)]
                sel = mv > 0.5
                sels.append(sel)
                cums.append(plsc.cumsum(sel.astype(jnp.int32)))
                pops.append(plsc.all_reduce_population_count(sel))
            base = cv
            for u in range(4):
                pos = base + cums[u] - 1
                ids = lax.iota(jnp.int32, 16) + (i * 4 + u) * 16
                okm = sels[u] & (pos < SPAD)
                plsc.store_scatter(sid_v, [pos], ids, mask=okm)
                base = base + pops[u]
            return base

        cntv = lax.fori_loop(0, N // 64, comp_step, jnp.zeros((16,), jnp.int32))
        cnt = cntv[0]
        pltpu.make_async_copy(vt_hbm, vt_v, csem).wait()
        pltpu.make_async_copy(beta_hbm, beta_v, bsem).wait()

        def gath_body(j, c):
            for u in range(2):
                off = (j * 2 + u) * 16
                posv = lax.iota(jnp.int32, 16) + off
                valid = posv < cnt
                idxv = jnp.where(valid, sid_v[pl.ds(off, 16)], 0)
                bk = plsc.load_gather(beta_v, [idxv])
                bs_v[pl.ds(off, 16)] = jnp.where(valid, bk, -1e9)
                for k in range(K):
                    kk = jnp.full((16,), k, jnp.int32)
                    vs_v[pl.ds(k * SPAD + off, 16)] = plsc.load_gather(
                        vt_v, [kk, idxv])
            return c

        lax.fori_loop(0, SPAD // 32, gath_body, 0)
        pltpu.sync_copy(vs_v, vsamp_hbm)
        pltpu.sync_copy(bs_v, bsamp_hbm)


_comp_call = pl.kernel(
    _comp_body,
    out_type=[
        jax.ShapeDtypeStruct((K * SPAD,), jnp.float32),
        jax.ShapeDtypeStruct((SPAD,), jnp.float32),
    ],
    mesh=_SC_MESH,
    scratch_types=[
        pltpu.VMEM((K, N), jnp.float32),
        pltpu.VMEM((N,), jnp.float32),
        pltpu.VMEM((N,), jnp.float32),
        pltpu.VMEM((SPAD,), jnp.int32),
        pltpu.VMEM((K * SPAD,), jnp.float32),
        pltpu.VMEM((SPAD,), jnp.float32),
        pltpu.SemaphoreType.DMA,
        pltpu.SemaphoreType.DMA,
    ],
    compiler_params=_SC_PARAMS,
)


def _sc_sqrt(x):
    b = plsc.bitcast(x, jnp.int32)
    y = plsc.bitcast((b >> 1) + jnp.int32(0x1FBD1DF5), jnp.float32)
    return 0.5 * (y + x / y)


def _edge_body(vt_hbm, beta_hbm, mask_hbm, ii_hbm, jj_hbm, acc_hbm,
               vt_v, beta_v, mask_v, ia_v, ja_v, ib_v, jb_v, si_v, sj_v, st_v,
               sia, sja, sib, sjb):
    wid = lax.axis_index("s") * 2 + lax.axis_index("c")
    tbase = wid * EPW

    def _start(buf_i, buf_j, sem_i, sem_j, ci):
        base = tbase + ci * ECH
        pltpu.make_async_copy(ii_hbm.at[pl.ds(base, ECH)], buf_i, sem_i).start()
        pltpu.make_async_copy(jj_hbm.at[pl.ds(base, ECH)], buf_j, sem_j).start()

    def _wait(buf_i, buf_j, sem_i, sem_j):
        pltpu.make_async_copy(ii_hbm.at[pl.ds(0, ECH)], buf_i, sem_i).wait()
        pltpu.make_async_copy(jj_hbm.at[pl.ds(0, ECH)], buf_j, sem_j).wait()

    U1 = 10

    def _pass1(ich, jch, step, ecv):
        sels, iis, jjs, cums, pops = [], [], [], [], []
        for u in range(U1):
            off = step * (16 * U1) + u * 16
            ii = ich[pl.ds(off, 16)]
            jj = jch[pl.ds(off, 16)]
            keep = (plsc.load_gather(mask_v, [ii])
                    * plsc.load_gather(mask_v, [jj]))
            sel = keep > 0.5
            iis.append(ii)
            jjs.append(jj)
            sels.append(sel)
            cums.append(plsc.cumsum(sel.astype(jnp.int32)))
            pops.append(plsc.all_reduce_population_count(sel))
        base = ecv
        for u in range(U1):
            pos = base + cums[u] - 1
            plsc.store_scatter(si_v, [pos], iis[u], mask=sels[u])
            plsc.store_scatter(sj_v, [pos], jjs[u], mask=sels[u])
            base = base + pops[u]
        return base

    def _pass2(vi, carry):
        ab, ad = carry
        off = vi * 16
        lane = lax.iota(jnp.int32, 16) + off
        ec = carry_ec[0]
        valid = lane < ec
        ii = jnp.where(valid, si_v[pl.ds(off, 16)], 0)
        jj = jnp.where(valid, sj_v[pl.ds(off, 16)], 0)
        bsum = plsc.load_gather(beta_v, [ii]) + plsc.load_gather(beta_v, [jj])
        sq = []
        for k in range(K):
            kk = jnp.full((16,), k, jnp.int32)
            d = (plsc.load_gather(vt_v, [kk, ii])
                 - plsc.load_gather(vt_v, [kk, jj]) + 1e-6)
            sq.append(d * d)
        while len(sq) > 1:
            sq = [x + y for x, y in zip(sq[0::2], sq[1::2])]
        vf = valid.astype(jnp.float32)
        return ab + vf * bsum, ad + vf * _sc_sqrt(sq[0])

    def _chunk(ich, jch, carry):
        ecv = lax.fori_loop(0, ECH // (16 * U1),
                            lambda it, c: _pass1(ich, jch, it, c),
                            jnp.zeros((16,), jnp.int32))
        ec = ecv[0]
        carry_ec[0] = ec
        nv2 = (ec + 15) // 16
        return lax.fori_loop(0, nv2, _pass2, carry)

    carry_ec = [jnp.int32(0)]
    _start(ia_v, ja_v, sia, sja, 0)
    pltpu.sync_copy(vt_hbm, vt_v)
    pltpu.sync_copy(beta_hbm, beta_v)
    pltpu.sync_copy(mask_hbm, mask_v)
    a0 = jnp.zeros((16,), jnp.float32)
    carry = (a0, a0)
    for ci in range(NCH):
        cur_i, cur_j = (ia_v, ja_v) if ci % 2 == 0 else (ib_v, jb_v)
        csi, csj = (sia, sja) if ci % 2 == 0 else (sib, sjb)
        nxt_i, nxt_j = (ib_v, jb_v) if ci % 2 == 0 else (ia_v, ja_v)
        nsi, nsj = (sib, sjb) if ci % 2 == 0 else (sia, sja)
        _wait(cur_i, cur_j, csi, csj)
        if ci + 1 < NCH:
            _start(nxt_i, nxt_j, nsi, nsj, ci + 1)
        carry = _chunk(cur_i, cur_j, carry)
    accb, accd = carry
    st_v[pl.ds(0, 16)] = accb
    st_v[pl.ds(16, 16)] = accd
    pltpu.sync_copy(st_v, acc_hbm.at[pl.ds(wid * 32, 32)])


_edge_call = pl.kernel(
    _edge_body,
    out_type=jax.ShapeDtypeStruct((NW * 32,), jnp.float32),
    mesh=_SC_MESH,
    scratch_types=[
        pltpu.VMEM((K, N), jnp.float32),
        pltpu.VMEM((N,), jnp.float32),
        pltpu.VMEM((N,), jnp.float32),
        pltpu.VMEM((ECH,), jnp.int32),
        pltpu.VMEM((ECH,), jnp.int32),
        pltpu.VMEM((ECH,), jnp.int32),
        pltpu.VMEM((ECH,), jnp.int32),
        pltpu.VMEM((ECH,), jnp.int32),
        pltpu.VMEM((ECH,), jnp.int32),
        pltpu.VMEM((32,), jnp.float32),
        pltpu.SemaphoreType.DMA,
        pltpu.SemaphoreType.DMA,
        pltpu.SemaphoreType.DMA,
        pltpu.SemaphoreType.DMA,
    ],
    compiler_params=_SC_PARAMS,
)


def _dense_body(vs_ref, bs_ref, a_ref, z1_ref):
    i = pl.program_id(0)
    at_full = vs_ref[...]
    bs = bs_ref[...]
    atr = vs_ref[:, pl.ds(i * RB, RB)]
    bsr = bs_ref[:, pl.ds(i * RB, RB)]

    ones_t = jnp.ones((1, SPAD), jnp.float32)
    cdims = (((0,), (0,)), ((), ()))
    nt = jnp.sum(at_full * at_full, axis=0, keepdims=True)
    rt = jnp.sum(at_full, axis=0, keepdims=True)
    nr = lax.dot_general(jnp.sum(atr * atr, axis=0, keepdims=True), ones_t,
                         cdims, preferred_element_type=jnp.float32)
    rr = lax.dot_general(jnp.sum(atr, axis=0, keepdims=True), ones_t,
                         cdims, preferred_element_type=jnp.float32)
    br = lax.dot_general(bsr, ones_t, cdims,
                         preferred_element_type=jnp.float32)
    p = lax.dot_general(atr, at_full, cdims,
                        preferred_element_type=jnp.float32)

    a = a_ref[0]
    sa = jnp.maximum(a, 0.0) + jnp.log(1.0 + jnp.exp(-jnp.abs(a)))
    d2 = nr + nt - 2.0 * p + 2e-6 * (rr - rt) + 8e-12
    dist = jnp.sqrt(jnp.maximum(d2, 0.0))
    mat = jnp.exp(br + bs - sa * dist)
    rowi = lax.broadcasted_iota(jnp.int32, (RB, SPAD), 0) + i * RB
    coli = lax.broadcasted_iota(jnp.int32, (RB, SPAD), 1)
    s_off = jnp.sum(jnp.where(rowi == coli, 0.0, mat))
    e1 = jnp.exp(jnp.float32(1.0))
    part = 0.5 * (e1 * e1) * s_off

    @pl.when(i == 0)
    def _():
        z1_ref[0, 0] = part

    @pl.when(i > 0)
    def _():
        z1_ref[0, 0] = z1_ref[0, 0] + part


_dense_call = pl.pallas_call(
    _dense_body,
    grid=(NRB,),
    in_specs=[
        pl.BlockSpec((K, SPAD), lambda i: (0, 0)),
        pl.BlockSpec((1, SPAD), lambda i: (0, 0)),
        pl.BlockSpec(memory_space=pltpu.SMEM),
    ],
    out_specs=pl.BlockSpec(memory_space=pltpu.SMEM),
    out_shape=jax.ShapeDtypeStruct((1, 1), jnp.float32),
)


def _comb_body(acc_ref, z1_ref, a_ref, out_ref):
    sb = jnp.sum(acc_ref[:, 0:16])
    sd = jnp.sum(acc_ref[:, 16:32])
    a = a_ref[0]
    sa = jnp.maximum(a, 0.0) + jnp.log(1.0 + jnp.exp(-jnp.abs(a)))
    out_ref[0, 0] = (sb - sa * sd) - z1_ref[0, 0]


_comb_call = pl.pallas_call(
    _comb_body,
    in_specs=[
        pl.BlockSpec((NW, 32), lambda: (0, 0)),
        pl.BlockSpec(memory_space=pltpu.SMEM),
        pl.BlockSpec(memory_space=pltpu.SMEM),
    ],
    out_specs=pl.BlockSpec(memory_space=pltpu.SMEM),
    out_shape=jax.ShapeDtypeStruct((1, 1), jnp.float32),
)


def kernel(sampling_weights, sparse_i_idx, sparse_j_idx, beta, a, Z, G):
    negg = -jax.random.gumbel(jax.random.key(123), (1, N), jnp.float32)
    maskf, vt = _prep_call(sampling_weights.reshape(1, N), negg, Z, G.T)
    mask1 = maskf.reshape(N)
    vsamp, bsamp = _comp_call(vt, beta, mask1)
    acc = _edge_call(vt, beta, mask1, sparse_i_idx, sparse_j_idx)
    z1 = _dense_call(vsamp.reshape(K, SPAD), bsamp.reshape(1, SPAD), a)
    out = _comb_call(acc.reshape(NW, 32), z1, a)
    return out[0, 0]
